# baseline (speedup 1.0000x reference)
import functools

import jax
import jax.numpy as jnp
from jax import lax
from jax.experimental import pallas as pl
from jax.experimental.pallas import tpu as pltpu
from jax.experimental.pallas import tpu_sc as plsc

B = 256
T = 197
D = 768
P = 16384
K = 8

_P_BLK = 2048


def _l2_normalize(x):
    sq = jnp.sum(x * x, axis=-1, keepdims=True)
    return x * lax.rsqrt(jnp.maximum(sq, 1e-12))


def _make_sim_body(nb):
    def _sim_body(q_ref, k_ref, o_ref, g_ref):
        s = lax.dot_general(q_ref[...], k_ref[...],
                            (((1,), (1,)), ((), ())),
                            preferred_element_type=jnp.float32)
        o_ref[...] = s
        g_ref[...] = jnp.max(s.reshape(nb, _P_BLK // 128, 128), axis=2)[None]
    return _sim_body


def _sim_call(q_norm, key_norm):
    nb = q_norm.shape[0]
    return pl.pallas_call(
        _make_sim_body(nb),
        grid=(P // _P_BLK,),
        in_specs=[
            pl.BlockSpec((nb, D), lambda j: (0, 0)),
            pl.BlockSpec((_P_BLK, D), lambda j: (j, 0)),
        ],
        out_specs=[
            pl.BlockSpec((nb, _P_BLK), lambda j: (0, j)),
            pl.BlockSpec((1, nb, _P_BLK // 128), lambda j: (j, 0, 0)),
        ],
        out_shape=[
            jax.ShapeDtypeStruct((nb, P), jnp.float32),
            jax.ShapeDtypeStruct((P // _P_BLK, nb, _P_BLK // 128),
                                 jnp.float32),
        ],
    )(q_norm, key_norm)


_NC, _NS = 2, 16
_NW = _NC * _NS
_NCHUNK = P // 16
_NEG_INF = float("-inf")
_BIG_I = 2**30

_G = 16
_NGRP = _NCHUNK // _G
_NGMAX = P // 128


def _sc_topk_row(row_v, g_all_v, rr, cv_v, ci_v, iota16):
    gb = rr * _NGMAX
    lane_max = g_all_v[pl.ds(gb, 16)]
    for c in range(1, _NGMAX // 16):
        lane_max = jnp.maximum(lane_max, g_all_v[pl.ds(gb + c * 16, 16)])
    s_max = plsc.sort_key_val(lane_max, lane_max)[0]
    thr = jnp.full((16,), 1.0, jnp.float32) * s_max[8]

    def p2(g, n):
        vs = [row_v[pl.ds((g * _G + j) * 16, 16)] for j in range(_G)]
        ms = [v >= thr for v in vs]
        any_m = ms[0]
        for j in range(1, _G):
            any_m = any_m | ms[j]
        hit = plsc.all_reduce_population_count(any_m)[0] > 0

        def slow(nn):
            for j in range(_G):
                cntj = plsc.all_reduce_population_count(ms[j])[0]

                def do(nj, j=j):
                    pos = nj + plsc.cumsum(ms[j].astype(jnp.int32)) - 1
                    plsc.store_scatter(cv_v, [pos], vs[j], mask=ms[j])
                    plsc.store_scatter(ci_v, [pos],
                                       iota16 + (g * _G + j) * 16, mask=ms[j])
                    return nj + cntj
                nn = lax.cond(cntj > 0, do, lambda nj: nj, nn)
            return nn
        return lax.cond(hit, slow, lambda nn: nn, n)
    n_cand = lax.fori_loop(0, _NGRP, p2, jnp.int32(0))
    nch = (n_cand + 15) // 16
    n_vec = jnp.full((16,), 1, jnp.int32) * n_cand

    res_v = jnp.zeros((16,), jnp.float32)
    res_i = jnp.zeros((16,), jnp.int32)
    for it in range(K):
        def scan(j, st):
            bv, bi, bp = st
            g = iota16 + j * 16
            v = jnp.where(g < n_vec, cv_v[pl.ds(j * 16, 16)], _NEG_INF)
            ids = ci_v[pl.ds(j * 16, 16)]
            better = (v > bv) | ((v == bv) & (ids < bi))
            return (jnp.where(better, v, bv),
                    jnp.where(better, ids, bi),
                    jnp.where(better, g, bp))
        bv, bi, bp = lax.fori_loop(
            0, nch, scan,
            (jnp.full((16,), _NEG_INF), jnp.full((16,), _BIG_I),
             jnp.full((16,), _BIG_I)))
        mval = jnp.full((16,), 1.0, jnp.float32) * lax.reduce_max(bv, (0,))
        sel = bv == mval
        mid = jnp.full((16,), 1, jnp.int32) * lax.reduce_min(
            jnp.where(sel, bi, _BIG_I), (0,))
        mpos = jnp.full((16,), 1, jnp.int32) * lax.reduce_min(
            jnp.where(sel & (bi == mid), bp, _BIG_I), (0,))
        res_v = jnp.where(iota16 == it, mval, res_v)
        res_i = jnp.where(iota16 == it, mid, res_i)
        plsc.store_scatter(cv_v, [mpos], jnp.full((16,), _NEG_INF),
                           mask=iota16 == 0)
    return res_v, res_i


def _build_sc_topk(nb):
    rpw = nb // _NW

    def body(sim_hbm, gflat_hbm, tv_hbm, ti_hbm, row_a, row_b, g_all_v,
             cv_v, ci_v, ov_v, oi_v, sem0, sem1):
        wid = lax.axis_index("s") * _NC + lax.axis_index("c")
        base = wid * rpw
        iota16 = lax.iota(jnp.int32, 16)
        rows = (row_a, row_b)
        sems = (sem0, sem1)

        pltpu.sync_copy(gflat_hbm.at[pl.ds(base * _NGMAX, rpw * _NGMAX)],
                        g_all_v)
        handle = pltpu.async_copy(sim_hbm.at[base], rows[0], sems[0])
        for rr in range(rpw):
            b = rr % 2
            handle.wait()
            if rr + 1 < rpw:
                handle = pltpu.async_copy(sim_hbm.at[base + (rr + 1)],
                                          rows[1 - b], sems[1 - b])
            res_v, res_i = _sc_topk_row(rows[b], g_all_v, rr, cv_v, ci_v,
                                        iota16)
            ov_v[rr] = res_v
            oi_v[rr] = res_i
        pltpu.sync_copy(ov_v, tv_hbm.at[pl.ds(base, rpw)])
        pltpu.sync_copy(oi_v, ti_hbm.at[pl.ds(base, rpw)])

    return pl.kernel(
        body,
        out_type=[jax.ShapeDtypeStruct((nb, 16), jnp.float32),
                  jax.ShapeDtypeStruct((nb, 16), jnp.int32)],
        mesh=plsc.VectorSubcoreMesh(core_axis_name="c", subcore_axis_name="s"),
        compiler_params=pltpu.CompilerParams(needs_layout_passes=False),
        scratch_types=[pltpu.VMEM((P,), jnp.float32),
                       pltpu.VMEM((P,), jnp.float32),
                       pltpu.VMEM((rpw * _NGMAX,), jnp.float32),
                       pltpu.VMEM((P + 16,), jnp.float32),
                       pltpu.VMEM((P + 16,), jnp.int32),
                       pltpu.VMEM((rpw, 16), jnp.float32),
                       pltpu.VMEM((rpw, 16), jnp.int32),
                       pltpu.SemaphoreType.DMA,
                       pltpu.SemaphoreType.DMA],
    )


_HB = B // 2


@jax.jit
def kernel(x_embed, prompt_key):
    q_norm = _l2_normalize(jnp.mean(x_embed, axis=1)).astype(jnp.bfloat16)
    key_norm = _l2_normalize(prompt_key).astype(jnp.bfloat16)

    topk_half = _build_sc_topk(_HB)
    sims, tvs, tis = [], [], []
    for h in range(2):
        sim_h, gmax_h = _sim_call(q_norm[h * _HB:(h + 1) * _HB], key_norm)
        gflat_h = gmax_h.transpose(1, 0, 2).reshape(-1)
        tv_h, ti_h = topk_half(sim_h, gflat_h)
        sims.append(sim_h)
        tvs.append(tv_h[:, :K])
        tis.append(ti_h[:, :K])

    sim = jnp.concatenate(sims, axis=0)
    return sim, jnp.concatenate(tvs, 0), jnp.concatenate(tis, 0)


# speedup vs baseline: 1.0886x; 1.0886x over previous
import functools

import jax
import jax.numpy as jnp
from jax import lax
from jax.experimental import pallas as pl
from jax.experimental.pallas import tpu as pltpu
from jax.experimental.pallas import tpu_sc as plsc

B = 256
T = 197
D = 768
P = 16384
K = 8

_P_BLK = 4096
_TK_BLK = 32


def _l2_normalize(x):
    sq = jnp.sum(x * x, axis=-1, keepdims=True)
    return x * lax.rsqrt(jnp.maximum(sq, 1e-12))


def _sim_body(q_ref, k_ref, o_ref, g_ref):
    s = lax.dot_general(q_ref[...], k_ref[...],
                        (((1,), (1,)), ((), ())),
                        preferred_element_type=jnp.float32)
    o_ref[...] = s
    g_ref[...] = jnp.max(s.reshape(B, _P_BLK // 128, 128), axis=2)[None]


_NC, _NS = 2, 16
_NW = _NC * _NS
_RPW = B // _NW
_NCHUNK = P // 16
_NEG_INF = float("-inf")
_BIG_I = 2**30


_G = 16
_NGRP = _NCHUNK // _G
_NGMAX = P // 128


def _sc_topk_row(row_v, g_all_v, rr, cv_v, ci_v, iota16):
    gb = rr * _NGMAX
    lane_max = g_all_v[pl.ds(gb, 16)]
    for c in range(1, _NGMAX // 16):
        lane_max = jnp.maximum(lane_max, g_all_v[pl.ds(gb + c * 16, 16)])
    s_max = plsc.sort_key_val(lane_max, lane_max)[0]
    thr = jnp.full((16,), 1.0, jnp.float32) * s_max[8]

    def p2(g, n):
        vs = [row_v[pl.ds((g * _G + j) * 16, 16)] for j in range(_G)]
        ms = [v >= thr for v in vs]
        any_m = ms[0]
        for j in range(1, _G):
            any_m = any_m | ms[j]
        hit = plsc.all_reduce_population_count(any_m)[0] > 0

        def slow(nn):
            for j in range(_G):
                cntj = plsc.all_reduce_population_count(ms[j])[0]

                def do(nj, j=j):
                    pos = nj + plsc.cumsum(ms[j].astype(jnp.int32)) - 1
                    plsc.store_scatter(cv_v, [pos], vs[j], mask=ms[j])
                    plsc.store_scatter(ci_v, [pos],
                                       iota16 + (g * _G + j) * 16, mask=ms[j])
                    return nj + cntj
                nn = lax.cond(cntj > 0, do, lambda nj: nj, nn)
            return nn
        return lax.cond(hit, slow, lambda nn: nn, n)
    n_cand = lax.fori_loop(0, _NGRP, p2, jnp.int32(0))
    nch = (n_cand + 15) // 16
    n_vec = jnp.full((16,), 1, jnp.int32) * n_cand

    res_v = jnp.zeros((16,), jnp.float32)
    res_i = jnp.zeros((16,), jnp.int32)
    for it in range(K):
        def scan(j, st):
            bv, bi, bp = st
            g = iota16 + j * 16
            v = jnp.where(g < n_vec, cv_v[pl.ds(j * 16, 16)], _NEG_INF)
            ids = ci_v[pl.ds(j * 16, 16)]
            better = (v > bv) | ((v == bv) & (ids < bi))
            return (jnp.where(better, v, bv),
                    jnp.where(better, ids, bi),
                    jnp.where(better, g, bp))
        bv, bi, bp = lax.fori_loop(
            0, nch, scan,
            (jnp.full((16,), _NEG_INF), jnp.full((16,), _BIG_I),
             jnp.full((16,), _BIG_I)))
        mval = jnp.full((16,), 1.0, jnp.float32) * lax.reduce_max(bv, (0,))
        sel = bv == mval
        mid = jnp.full((16,), 1, jnp.int32) * lax.reduce_min(
            jnp.where(sel, bi, _BIG_I), (0,))
        mpos = jnp.full((16,), 1, jnp.int32) * lax.reduce_min(
            jnp.where(sel & (bi == mid), bp, _BIG_I), (0,))
        res_v = jnp.where(iota16 == it, mval, res_v)
        res_i = jnp.where(iota16 == it, mid, res_i)
        plsc.store_scatter(cv_v, [mpos], jnp.full((16,), _NEG_INF),
                           mask=iota16 == 0)
    return res_v, res_i


def _sc_topk_body(sim_hbm, gflat_hbm, tv_hbm, ti_hbm, row_a, row_b, g_all_v,
                  cv_v, ci_v, ov_v, oi_v, sem0, sem1):
    wid = lax.axis_index("s") * _NC + lax.axis_index("c")
    base = wid * _RPW
    iota16 = lax.iota(jnp.int32, 16)
    rows = (row_a, row_b)
    sems = (sem0, sem1)

    pltpu.sync_copy(gflat_hbm.at[pl.ds(base * _NGMAX, _RPW * _NGMAX)], g_all_v)
    handle = pltpu.async_copy(sim_hbm.at[base], rows[0], sems[0])
    for rr in range(_RPW):
        b = rr % 2
        handle.wait()
        if rr + 1 < _RPW:
            handle = pltpu.async_copy(sim_hbm.at[base + (rr + 1)],
                                      rows[1 - b], sems[1 - b])
        res_v, res_i = _sc_topk_row(rows[b], g_all_v, rr, cv_v, ci_v, iota16)
        ov_v[rr] = res_v
        oi_v[rr] = res_i
    pltpu.sync_copy(ov_v, tv_hbm.at[pl.ds(base, _RPW)])
    pltpu.sync_copy(oi_v, ti_hbm.at[pl.ds(base, _RPW)])


@functools.partial(
    pl.kernel,
    out_type=[jax.ShapeDtypeStruct((B, 16), jnp.float32),
              jax.ShapeDtypeStruct((B, 16), jnp.int32)],
    mesh=plsc.VectorSubcoreMesh(core_axis_name="c", subcore_axis_name="s"),
    compiler_params=pltpu.CompilerParams(needs_layout_passes=False),
    scratch_types=[pltpu.VMEM((P,), jnp.float32),
                   pltpu.VMEM((P,), jnp.float32),
                   pltpu.VMEM((_RPW * _NGMAX,), jnp.float32),
                   pltpu.VMEM((P + 16,), jnp.float32),
                   pltpu.VMEM((P + 16,), jnp.int32),
                   pltpu.VMEM((_RPW, 16), jnp.float32),
                   pltpu.VMEM((_RPW, 16), jnp.int32),
                   pltpu.SemaphoreType.DMA,
                   pltpu.SemaphoreType.DMA],
)
def _sc_topk(sim_hbm, gflat_hbm, tv_hbm, ti_hbm, row_a, row_b, g_all_v,
             cv_v, ci_v, ov_v, oi_v, sem0, sem1):
    _sc_topk_body(sim_hbm, gflat_hbm, tv_hbm, ti_hbm, row_a, row_b, g_all_v,
                  cv_v, ci_v, ov_v, oi_v, sem0, sem1)


@jax.jit
def kernel(x_embed, prompt_key):
    q_norm = _l2_normalize(jnp.mean(x_embed, axis=1)).astype(jnp.bfloat16)
    key_norm = _l2_normalize(prompt_key).astype(jnp.bfloat16)

    sim, gmax = pl.pallas_call(
        _sim_body,
        grid=(P // _P_BLK,),
        in_specs=[
            pl.BlockSpec((B, D), lambda j: (0, 0)),
            pl.BlockSpec((_P_BLK, D), lambda j: (j, 0)),
        ],
        out_specs=[
            pl.BlockSpec((B, _P_BLK), lambda j: (0, j)),
            pl.BlockSpec((1, B, _P_BLK // 128), lambda j: (j, 0, 0)),
        ],
        out_shape=[
            jax.ShapeDtypeStruct((B, P), jnp.float32),
            jax.ShapeDtypeStruct((P // _P_BLK, B, _P_BLK // 128), jnp.float32),
        ],
    )(q_norm, key_norm)

    gflat = gmax.transpose(1, 0, 2).reshape(-1)
    tv16, ti16 = _sc_topk(sim, gflat)
    return sim, tv16[:, :K], ti16[:, :K]


# speedup vs baseline: 1.0905x; 1.0018x over previous
import functools

import jax
import jax.numpy as jnp
from jax import lax
from jax.experimental import pallas as pl
from jax.experimental.pallas import tpu as pltpu
from jax.experimental.pallas import tpu_sc as plsc

B = 256
T = 197
D = 768
P = 16384
K = 8

_P_BLK = 8192
_TK_BLK = 32


def _l2_normalize(x):
    sq = jnp.sum(x * x, axis=-1, keepdims=True)
    return x * lax.rsqrt(jnp.maximum(sq, 1e-12))


def _sim_body(q_ref, k_ref, o_ref, g_ref):
    s = lax.dot_general(q_ref[...], k_ref[...],
                        (((1,), (1,)), ((), ())),
                        preferred_element_type=jnp.float32)
    o_ref[...] = s
    g_ref[...] = jnp.max(s.reshape(B, _P_BLK // 128, 128), axis=2)[None]


_NC, _NS = 2, 16
_NW = _NC * _NS
_RPW = B // _NW
_NCHUNK = P // 16
_NEG_INF = float("-inf")
_BIG_I = 2**30


_G = 16
_NGRP = _NCHUNK // _G
_NGMAX = P // 128


def _sc_topk_row(row_v, g_all_v, rr, cv_v, ci_v, iota16):
    gb = rr * _NGMAX
    lane_max = g_all_v[pl.ds(gb, 16)]
    for c in range(1, _NGMAX // 16):
        lane_max = jnp.maximum(lane_max, g_all_v[pl.ds(gb + c * 16, 16)])
    s_max = plsc.sort_key_val(lane_max, lane_max)[0]
    thr = jnp.full((16,), 1.0, jnp.float32) * s_max[8]

    def p2(g, n):
        vs = [row_v[pl.ds((g * _G + j) * 16, 16)] for j in range(_G)]
        ms = [v >= thr for v in vs]
        any_m = ms[0]
        for j in range(1, _G):
            any_m = any_m | ms[j]
        hit = plsc.all_reduce_population_count(any_m)[0] > 0

        def slow(nn):
            for j in range(_G):
                cntj = plsc.all_reduce_population_count(ms[j])[0]

                def do(nj, j=j):
                    pos = nj + plsc.cumsum(ms[j].astype(jnp.int32)) - 1
                    plsc.store_scatter(cv_v, [pos], vs[j], mask=ms[j])
                    plsc.store_scatter(ci_v, [pos],
                                       iota16 + (g * _G + j) * 16, mask=ms[j])
                    return nj + cntj
                nn = lax.cond(cntj > 0, do, lambda nj: nj, nn)
            return nn
        return lax.cond(hit, slow, lambda nn: nn, n)
    n_cand = lax.fori_loop(0, _NGRP, p2, jnp.int32(0))
    nch = (n_cand + 15) // 16
    n_vec = jnp.full((16,), 1, jnp.int32) * n_cand

    res_v = jnp.zeros((16,), jnp.float32)
    res_i = jnp.zeros((16,), jnp.int32)
    for it in range(K):
        def scan(j, st):
            bv, bi, bp = st
            g = iota16 + j * 16
            v = jnp.where(g < n_vec, cv_v[pl.ds(j * 16, 16)], _NEG_INF)
            ids = ci_v[pl.ds(j * 16, 16)]
            better = (v > bv) | ((v == bv) & (ids < bi))
            return (jnp.where(better, v, bv),
                    jnp.where(better, ids, bi),
                    jnp.where(better, g, bp))
        bv, bi, bp = lax.fori_loop(
            0, nch, scan,
            (jnp.full((16,), _NEG_INF), jnp.full((16,), _BIG_I),
             jnp.full((16,), _BIG_I)))
        mval = jnp.full((16,), 1.0, jnp.float32) * lax.reduce_max(bv, (0,))
        sel = bv == mval
        mid = jnp.full((16,), 1, jnp.int32) * lax.reduce_min(
            jnp.where(sel, bi, _BIG_I), (0,))
        mpos = jnp.full((16,), 1, jnp.int32) * lax.reduce_min(
            jnp.where(sel & (bi == mid), bp, _BIG_I), (0,))
        res_v = jnp.where(iota16 == it, mval, res_v)
        res_i = jnp.where(iota16 == it, mid, res_i)
        plsc.store_scatter(cv_v, [mpos], jnp.full((16,), _NEG_INF),
                           mask=iota16 == 0)
    return res_v, res_i


def _sc_topk_body(sim_hbm, gflat_hbm, tv_hbm, ti_hbm, row_a, row_b, g_all_v,
                  cv_v, ci_v, ov_v, oi_v, sem0, sem1):
    wid = lax.axis_index("s") * _NC + lax.axis_index("c")
    base = wid * _RPW
    iota16 = lax.iota(jnp.int32, 16)
    rows = (row_a, row_b)
    sems = (sem0, sem1)

    pltpu.sync_copy(gflat_hbm.at[pl.ds(base * _NGMAX, _RPW * _NGMAX)], g_all_v)
    handle = pltpu.async_copy(sim_hbm.at[base], rows[0], sems[0])
    for rr in range(_RPW):
        b = rr % 2
        handle.wait()
        if rr + 1 < _RPW:
            handle = pltpu.async_copy(sim_hbm.at[base + (rr + 1)],
                                      rows[1 - b], sems[1 - b])
        res_v, res_i = _sc_topk_row(rows[b], g_all_v, rr, cv_v, ci_v, iota16)
        ov_v[rr] = res_v
        oi_v[rr] = res_i
    pltpu.sync_copy(ov_v, tv_hbm.at[pl.ds(base, _RPW)])
    pltpu.sync_copy(oi_v, ti_hbm.at[pl.ds(base, _RPW)])


@functools.partial(
    pl.kernel,
    out_type=[jax.ShapeDtypeStruct((B, 16), jnp.float32),
              jax.ShapeDtypeStruct((B, 16), jnp.int32)],
    mesh=plsc.VectorSubcoreMesh(core_axis_name="c", subcore_axis_name="s"),
    compiler_params=pltpu.CompilerParams(needs_layout_passes=False),
    scratch_types=[pltpu.VMEM((P,), jnp.float32),
                   pltpu.VMEM((P,), jnp.float32),
                   pltpu.VMEM((_RPW * _NGMAX,), jnp.float32),
                   pltpu.VMEM((P + 16,), jnp.float32),
                   pltpu.VMEM((P + 16,), jnp.int32),
                   pltpu.VMEM((_RPW, 16), jnp.float32),
                   pltpu.VMEM((_RPW, 16), jnp.int32),
                   pltpu.SemaphoreType.DMA,
                   pltpu.SemaphoreType.DMA],
)
def _sc_topk(sim_hbm, gflat_hbm, tv_hbm, ti_hbm, row_a, row_b, g_all_v,
             cv_v, ci_v, ov_v, oi_v, sem0, sem1):
    _sc_topk_body(sim_hbm, gflat_hbm, tv_hbm, ti_hbm, row_a, row_b, g_all_v,
                  cv_v, ci_v, ov_v, oi_v, sem0, sem1)


@jax.jit
def kernel(x_embed, prompt_key):
    q_norm = _l2_normalize(jnp.mean(x_embed, axis=1)).astype(jnp.bfloat16)
    key_norm = _l2_normalize(prompt_key).astype(jnp.bfloat16)

    sim, gmax = pl.pallas_call(
        _sim_body,
        grid=(P // _P_BLK,),
        in_specs=[
            pl.BlockSpec((B, D), lambda j: (0, 0)),
            pl.BlockSpec((_P_BLK, D), lambda j: (j, 0)),
        ],
        out_specs=[
            pl.BlockSpec((B, _P_BLK), lambda j: (0, j)),
            pl.BlockSpec((1, B, _P_BLK // 128), lambda j: (j, 0, 0)),
        ],
        out_shape=[
            jax.ShapeDtypeStruct((B, P), jnp.float32),
            jax.ShapeDtypeStruct((P // _P_BLK, B, _P_BLK // 128), jnp.float32),
        ],
    )(q_norm, key_norm)

    gflat = gmax.transpose(1, 0, 2).reshape(-1)
    tv16, ti16 = _sc_topk(sim, gflat)
    return sim, tv16[:, :K], ti16[:, :K]


# speedup vs baseline: 1.0926x; 1.0019x over previous
import functools

import jax
import jax.numpy as jnp
from jax import lax
from jax.experimental import pallas as pl
from jax.experimental.pallas import tpu as pltpu
from jax.experimental.pallas import tpu_sc as plsc

B = 256
T = 197
D = 768
P = 16384
K = 8

_P_BLK = 8192


def _l2_normalize(x):
    sq = jnp.sum(x * x, axis=-1, keepdims=True)
    return x * lax.rsqrt(jnp.maximum(sq, 1e-12))


def _sim_body(q_ref, k_ref, o_ref, g_ref):
    s = lax.dot_general(q_ref[...], k_ref[...],
                        (((1,), (1,)), ((), ())),
                        preferred_element_type=jnp.float32)
    o_ref[...] = s
    g_ref[...] = jnp.max(s.reshape(B, _P_BLK // 128, 128), axis=2)[None]


_NC, _NS = 2, 16
_NW = _NC * _NS
_RPW = B // _NW
_NCHUNK = P // 16
_NEG_INF = float("-inf")
_BIG_I = 2**30


_G = 16
_NGRP = _NCHUNK // _G
_NGMAX = P // 128


def _sc_topk_row(row_v, g_all_v, rr, cv_v, ci_v, iota16):
    gb = rr * _NGMAX
    lane_max = g_all_v[pl.ds(gb, 16)]
    for c in range(1, _NGMAX // 16):
        lane_max = jnp.maximum(lane_max, g_all_v[pl.ds(gb + c * 16, 16)])
    s_max = plsc.sort_key_val(lane_max, lane_max)[0]
    thr = jnp.full((16,), 1.0, jnp.float32) * s_max[8]

    def p2(g, n):
        vs = [row_v[pl.ds((g * _G + j) * 16, 16)] for j in range(_G)]
        ms = [v >= thr for v in vs]
        any_m = ms[0]
        for j in range(1, _G):
            any_m = any_m | ms[j]
        hit = plsc.all_reduce_population_count(any_m)[0] > 0

        def slow(nn):
            for j in range(_G):
                cntj = plsc.all_reduce_population_count(ms[j])[0]

                def do(nj, j=j):
                    pos = nj + plsc.cumsum(ms[j].astype(jnp.int32)) - 1
                    plsc.store_scatter(cv_v, [pos], vs[j], mask=ms[j])
                    plsc.store_scatter(ci_v, [pos],
                                       iota16 + (g * _G + j) * 16, mask=ms[j])
                    return nj + cntj
                nn = lax.cond(cntj > 0, do, lambda nj: nj, nn)
            return nn
        return lax.cond(hit, slow, lambda nn: nn, n)
    n_cand = lax.fori_loop(0, _NGRP, p2, jnp.int32(0))
    nch = (n_cand + 15) // 16
    n_vec = jnp.full((16,), 1, jnp.int32) * n_cand

    res_v = jnp.zeros((16,), jnp.float32)
    res_i = jnp.zeros((16,), jnp.int32)
    for it in range(K):
        def scan(j, st):
            bv, bi, bp = st
            g = iota16 + j * 16
            v = jnp.where(g < n_vec, cv_v[pl.ds(j * 16, 16)], _NEG_INF)
            ids = ci_v[pl.ds(j * 16, 16)]
            better = (v > bv) | ((v == bv) & (ids < bi))
            return (jnp.where(better, v, bv),
                    jnp.where(better, ids, bi),
                    jnp.where(better, g, bp))
        bv, bi, bp = lax.fori_loop(
            0, nch, scan,
            (jnp.full((16,), _NEG_INF), jnp.full((16,), _BIG_I),
             jnp.full((16,), _BIG_I)))
        mval = jnp.full((16,), 1.0, jnp.float32) * lax.reduce_max(bv, (0,))
        sel = bv == mval
        mid = jnp.full((16,), 1, jnp.int32) * lax.reduce_min(
            jnp.where(sel, bi, _BIG_I), (0,))
        mpos = jnp.full((16,), 1, jnp.int32) * lax.reduce_min(
            jnp.where(sel & (bi == mid), bp, _BIG_I), (0,))
        res_v = jnp.where(iota16 == it, mval, res_v)
        res_i = jnp.where(iota16 == it, mid, res_i)
        plsc.store_scatter(cv_v, [mpos], jnp.full((16,), _NEG_INF),
                           mask=iota16 == 0)
    return res_v, res_i


def _sc_topk_body(sim_hbm, gflat_hbm, tv_hbm, ti_hbm, row_a, row_b, g_all_v,
                  cv_v, ci_v, ov_v, oi_v, sem0, sem1):
    wid = lax.axis_index("s") * _NC + lax.axis_index("c")
    base = wid * _RPW
    iota16 = lax.iota(jnp.int32, 16)
    rows = (row_a, row_b)
    sems = (sem0, sem1)

    pltpu.sync_copy(gflat_hbm.at[pl.ds(base * _NGMAX, _RPW * _NGMAX)], g_all_v)
    handle = pltpu.async_copy(sim_hbm.at[base], rows[0], sems[0])
    for rr in range(_RPW):
        b = rr % 2
        handle.wait()
        if rr + 1 < _RPW:
            handle = pltpu.async_copy(sim_hbm.at[base + (rr + 1)],
                                      rows[1 - b], sems[1 - b])
        res_v, res_i = _sc_topk_row(rows[b], g_all_v, rr, cv_v, ci_v, iota16)
        ov_v[rr] = res_v
        oi_v[rr] = res_i
    pltpu.sync_copy(ov_v, tv_hbm.at[pl.ds(base, _RPW)])
    pltpu.sync_copy(oi_v, ti_hbm.at[pl.ds(base, _RPW)])


@functools.partial(
    pl.kernel,
    out_type=[jax.ShapeDtypeStruct((B, 16), jnp.float32),
              jax.ShapeDtypeStruct((B, 16), jnp.int32)],
    mesh=plsc.VectorSubcoreMesh(core_axis_name="c", subcore_axis_name="s"),
    compiler_params=pltpu.CompilerParams(needs_layout_passes=False),
    scratch_types=[pltpu.VMEM((P,), jnp.float32),
                   pltpu.VMEM((P,), jnp.float32),
                   pltpu.VMEM((_RPW * _NGMAX,), jnp.float32),
                   pltpu.VMEM((P + 16,), jnp.float32),
                   pltpu.VMEM((P + 16,), jnp.int32),
                   pltpu.VMEM((_RPW, 16), jnp.float32),
                   pltpu.VMEM((_RPW, 16), jnp.int32),
                   pltpu.SemaphoreType.DMA,
                   pltpu.SemaphoreType.DMA],
)
def _sc_topk(sim_hbm, gflat_hbm, tv_hbm, ti_hbm, row_a, row_b, g_all_v,
             cv_v, ci_v, ov_v, oi_v, sem0, sem1):
    _sc_topk_body(sim_hbm, gflat_hbm, tv_hbm, ti_hbm, row_a, row_b, g_all_v,
                  cv_v, ci_v, ov_v, oi_v, sem0, sem1)


@jax.jit
def kernel(x_embed, prompt_key):
    q_norm = _l2_normalize(jnp.mean(x_embed, axis=1)).astype(jnp.bfloat16)
    key_norm = _l2_normalize(prompt_key).astype(jnp.bfloat16)

    sim, gmax = pl.pallas_call(
        _sim_body,
        grid=(P // _P_BLK,),
        in_specs=[
            pl.BlockSpec((B, D), lambda j: (0, 0)),
            pl.BlockSpec((_P_BLK, D), lambda j: (j, 0)),
        ],
        out_specs=[
            pl.BlockSpec((B, _P_BLK), lambda j: (0, j)),
            pl.BlockSpec((1, B, _P_BLK // 128), lambda j: (j, 0, 0)),
        ],
        out_shape=[
            jax.ShapeDtypeStruct((B, P), jnp.float32),
            jax.ShapeDtypeStruct((P // _P_BLK, B, _P_BLK // 128), jnp.float32),
        ],
    )(q_norm, key_norm)

    gflat = gmax.transpose(1, 0, 2).reshape(-1)
    tv16, ti16 = _sc_topk(sim, gflat)
    return sim, tv16[:, :K], ti16[:, :K]
